# Initial kernel scaffold; baseline (speedup 1.0000x reference)
import jax
import jax.numpy as jnp
from jax.experimental import pallas as pl


def kernel(x, weight, bias):
    raise NotImplementedError("write your pallas kernel here")


# speedup vs baseline: 1.2166x; 1.2166x over previous
import functools

import jax
import jax.numpy as jnp
from jax.experimental import pallas as pl
from jax.experimental.pallas import tpu as pltpu

TOP_K = 8
N_EXPERTS = 64
NEG_BIG = -1e30


def _gate_kernel(x_ref, wt_ref, b_ref, idx_ref, w_ref):
    logits = jnp.dot(x_ref[:], wt_ref[:], preferred_element_type=jnp.float32)
    logits = logits + b_ref[:]
    rows = logits.shape[0]
    lane = jax.lax.broadcasted_iota(jnp.int32, (rows, N_EXPERTS), 1)

    e = jnp.exp(logits - jnp.max(logits, axis=-1, keepdims=True))
    scores = e / jnp.sum(e, axis=-1, keepdims=True)

    vals = scores
    top_vals = []
    top_idx = []
    for _ in range(TOP_K):
        m = jnp.max(vals, axis=-1, keepdims=True)
        i = jnp.min(jnp.where(vals == m, lane, N_EXPERTS), axis=-1, keepdims=True)
        top_vals.append(m)
        top_idx.append(i)
        vals = jnp.where(lane == i, NEG_BIG, vals)

    tv = jnp.concatenate(top_vals, axis=-1)
    ti = jnp.concatenate(top_idx, axis=-1)
    w = tv / (jnp.sum(tv, axis=-1, keepdims=True) + 1e-20)
    idx_ref[:] = ti
    w_ref[:] = w


@functools.partial(jax.jit, static_argnames=())
def kernel(x, weight, bias):
    bsz, seq_len, h = x.shape
    n_rows = bsz * seq_len
    xf = x.reshape(n_rows, h)
    wt = weight.T
    b = bias.reshape(1, N_EXPERTS)

    block_rows = 1024
    grid = (n_rows // block_rows,)

    idx, w = pl.pallas_call(
        _gate_kernel,
        grid=grid,
        in_specs=[
            pl.BlockSpec((block_rows, h), lambda i: (i, 0)),
            pl.BlockSpec((h, N_EXPERTS), lambda i: (0, 0)),
            pl.BlockSpec((1, N_EXPERTS), lambda i: (0, 0)),
        ],
        out_specs=[
            pl.BlockSpec((block_rows, TOP_K), lambda i: (i, 0)),
            pl.BlockSpec((block_rows, TOP_K), lambda i: (i, 0)),
        ],
        out_shape=[
            jax.ShapeDtypeStruct((n_rows, TOP_K), jnp.int32),
            jax.ShapeDtypeStruct((n_rows, TOP_K), jnp.float32),
        ],
        compiler_params=pltpu.CompilerParams(
            dimension_semantics=("arbitrary",),
        ),
    )(xf, wt, b)

    aux_loss = jnp.asarray(0.0, dtype=jnp.float32)
    return (idx, w, aux_loss)


# speedup vs baseline: 1.4630x; 1.2025x over previous
import functools

import jax
import jax.numpy as jnp
from jax.experimental import pallas as pl
from jax.experimental.pallas import tpu as pltpu

TOP_K = 8
N_EXPERTS = 64
NEG_BIG = -1e30


def _gate_kernel(x_ref, wt_ref, b_ref, idx_ref, w_ref):
    logits = jnp.dot(x_ref[:], wt_ref[:], preferred_element_type=jnp.float32)
    logits = logits + b_ref[:]
    rows = logits.shape[0]
    lane = jax.lax.broadcasted_iota(jnp.int32, (rows, N_EXPERTS), 1)

    e = jnp.exp(logits - jnp.max(logits, axis=-1, keepdims=True))
    scores = e / jnp.sum(e, axis=-1, keepdims=True)

    sb = jax.lax.bitcast_convert_type(scores, jnp.int32)
    keys = (sb & jnp.int32(-64)) | (jnp.int32(N_EXPERTS - 1) - lane)

    top_keys = []
    for _ in range(TOP_K):
        m = jnp.max(keys, axis=-1, keepdims=True)
        top_keys.append(m)
        keys = jnp.where(keys == m, jnp.int32(-1), keys)

    tk = jnp.concatenate(top_keys, axis=-1)
    ti = jnp.int32(N_EXPERTS - 1) - (tk & jnp.int32(N_EXPERTS - 1))
    tv = jax.lax.bitcast_convert_type(tk & jnp.int32(-64), jnp.float32)
    w = tv / (jnp.sum(tv, axis=-1, keepdims=True) + 1e-20)
    idx_ref[:] = ti
    w_ref[:] = w


@functools.partial(jax.jit, static_argnames=())
def kernel(x, weight, bias):
    bsz, seq_len, h = x.shape
    n_rows = bsz * seq_len
    xf = x.reshape(n_rows, h)
    wt = weight.T
    b = bias.reshape(1, N_EXPERTS)

    block_rows = 1024
    grid = (n_rows // block_rows,)

    idx, w = pl.pallas_call(
        _gate_kernel,
        grid=grid,
        in_specs=[
            pl.BlockSpec((block_rows, h), lambda i: (i, 0)),
            pl.BlockSpec((h, N_EXPERTS), lambda i: (0, 0)),
            pl.BlockSpec((1, N_EXPERTS), lambda i: (0, 0)),
        ],
        out_specs=[
            pl.BlockSpec((block_rows, TOP_K), lambda i: (i, 0)),
            pl.BlockSpec((block_rows, TOP_K), lambda i: (i, 0)),
        ],
        out_shape=[
            jax.ShapeDtypeStruct((n_rows, TOP_K), jnp.int32),
            jax.ShapeDtypeStruct((n_rows, TOP_K), jnp.float32),
        ],
        compiler_params=pltpu.CompilerParams(
            dimension_semantics=("arbitrary",),
        ),
    )(xf, wt, b)

    aux_loss = jnp.asarray(0.0, dtype=jnp.float32)
    return (idx, w, aux_loss)


# speedup vs baseline: 1.6650x; 1.1380x over previous
import functools

import jax
import jax.numpy as jnp
from jax.experimental import pallas as pl
from jax.experimental.pallas import tpu as pltpu

TOP_K = 8
N_EXPERTS = 64
NEG_BIG = -1e30
BIAS = 0x10000000


def _gate_kernel(x_ref, wt_ref, b_ref, idx_ref, w_ref):
    logits = jnp.dot(x_ref[:], wt_ref[:], preferred_element_type=jnp.float32)
    logits = logits + b_ref[:]
    rows = logits.shape[0]
    lane = jax.lax.broadcasted_iota(jnp.int32, (rows, N_EXPERTS), 1)

    e = jnp.exp(logits - jnp.max(logits, axis=-1, keepdims=True))
    scores = e / jnp.sum(e, axis=-1, keepdims=True)

    sb = jax.lax.bitcast_convert_type(scores, jnp.int32)
    ikeys = ((sb & jnp.int32(-64)) | (jnp.int32(N_EXPERTS - 1) - lane)) + BIAS
    keys = jax.lax.bitcast_convert_type(ikeys, jnp.float32)

    top_keys = []
    for _ in range(TOP_K):
        m = jnp.max(keys, axis=-1, keepdims=True)
        top_keys.append(m)
        keys = jnp.where(keys == m, jnp.float32(-1.0), keys)

    tkf = jnp.concatenate(top_keys, axis=-1)
    tk = jax.lax.bitcast_convert_type(tkf, jnp.int32) - BIAS
    ti = jnp.int32(N_EXPERTS - 1) - (tk & jnp.int32(N_EXPERTS - 1))
    tv = jax.lax.bitcast_convert_type(tk & jnp.int32(-64), jnp.float32)
    w = tv / (jnp.sum(tv, axis=-1, keepdims=True) + 1e-20)
    idx_ref[:] = ti
    w_ref[:] = w


@functools.partial(jax.jit, static_argnames=())
def kernel(x, weight, bias):
    bsz, seq_len, h = x.shape
    n_rows = bsz * seq_len
    xf = x.reshape(n_rows, h)
    wt = weight.T
    b = bias.reshape(1, N_EXPERTS)

    block_rows = 1024
    grid = (n_rows // block_rows,)

    idx, w = pl.pallas_call(
        _gate_kernel,
        grid=grid,
        in_specs=[
            pl.BlockSpec((block_rows, h), lambda i: (i, 0)),
            pl.BlockSpec((h, N_EXPERTS), lambda i: (0, 0)),
            pl.BlockSpec((1, N_EXPERTS), lambda i: (0, 0)),
        ],
        out_specs=[
            pl.BlockSpec((block_rows, TOP_K), lambda i: (i, 0)),
            pl.BlockSpec((block_rows, TOP_K), lambda i: (i, 0)),
        ],
        out_shape=[
            jax.ShapeDtypeStruct((n_rows, TOP_K), jnp.int32),
            jax.ShapeDtypeStruct((n_rows, TOP_K), jnp.float32),
        ],
        compiler_params=pltpu.CompilerParams(
            dimension_semantics=("arbitrary",),
        ),
    )(xf, wt, b)

    aux_loss = jnp.asarray(0.0, dtype=jnp.float32)
    return (idx, w, aux_loss)


# speedup vs baseline: 1.7474x; 1.0495x over previous
import functools

import jax
import jax.numpy as jnp
from jax.experimental import pallas as pl
from jax.experimental.pallas import tpu as pltpu

TOP_K = 8
N_EXPERTS = 64
NEG_BIG = -1e30
BIAS = 0x10000000


def _gate_kernel(x_ref, wt_ref, b_ref, idx_ref, w_ref):
    logits = jnp.dot(x_ref[:], wt_ref[:], preferred_element_type=jnp.float32)
    logits = logits + b_ref[:]
    rows = logits.shape[0]
    lane = jax.lax.broadcasted_iota(jnp.int32, (rows, N_EXPERTS), 1)

    e = jnp.exp(logits - jnp.max(logits, axis=-1, keepdims=True))
    scores = e / jnp.sum(e, axis=-1, keepdims=True)

    sb = jax.lax.bitcast_convert_type(scores, jnp.int32)
    ikeys = ((sb & jnp.int32(-64)) | (jnp.int32(N_EXPERTS - 1) - lane)) + BIAS
    keys = jax.lax.bitcast_convert_type(ikeys, jnp.float32)

    top_keys = []
    for _ in range(TOP_K):
        m = jnp.max(keys, axis=-1, keepdims=True)
        top_keys.append(m)
        keys = jnp.where(keys == m, jnp.float32(-1.0), keys)

    tkf = jnp.concatenate(top_keys, axis=-1)
    tk = jax.lax.bitcast_convert_type(tkf, jnp.int32) - BIAS
    ti = jnp.int32(N_EXPERTS - 1) - (tk & jnp.int32(N_EXPERTS - 1))
    tv = jax.lax.bitcast_convert_type(tk & jnp.int32(-64), jnp.float32)
    w = tv / (jnp.sum(tv, axis=-1, keepdims=True) + 1e-20)
    idx_ref[:] = ti
    w_ref[:] = w


@functools.partial(jax.jit, static_argnames=())
def kernel(x, weight, bias):
    bsz, seq_len, h = x.shape
    n_rows = bsz * seq_len
    xf = x.reshape(n_rows, h)
    wt = weight.T
    b = bias.reshape(1, N_EXPERTS)

    block_rows = 2048
    grid = (n_rows // block_rows,)

    idx, w = pl.pallas_call(
        _gate_kernel,
        grid=grid,
        in_specs=[
            pl.BlockSpec((block_rows, h), lambda i: (i, 0)),
            pl.BlockSpec((h, N_EXPERTS), lambda i: (0, 0)),
            pl.BlockSpec((1, N_EXPERTS), lambda i: (0, 0)),
        ],
        out_specs=[
            pl.BlockSpec((block_rows, TOP_K), lambda i: (i, 0)),
            pl.BlockSpec((block_rows, TOP_K), lambda i: (i, 0)),
        ],
        out_shape=[
            jax.ShapeDtypeStruct((n_rows, TOP_K), jnp.int32),
            jax.ShapeDtypeStruct((n_rows, TOP_K), jnp.float32),
        ],
        compiler_params=pltpu.CompilerParams(
            dimension_semantics=("arbitrary",),
        ),
    )(xf, wt, b)

    aux_loss = jnp.asarray(0.0, dtype=jnp.float32)
    return (idx, w, aux_loss)
